# baseline (speedup 1.0000x reference)
import jax
import jax.numpy as jnp
from jax.experimental import pallas as pl
from jax.experimental.pallas import tpu as pltpu

_N = 1_000_000


def _copy_body(in_hbm, out_hbm, sem):
    copy = pltpu.make_async_copy(in_hbm, out_hbm, sem)
    copy.start()
    copy.wait()


def kernel(goal_logits):
    return pl.pallas_call(
        _copy_body,
        out_shape=jax.ShapeDtypeStruct((_N,), jnp.float32),
        in_specs=[pl.BlockSpec(memory_space=pl.ANY)],
        out_specs=pl.BlockSpec(memory_space=pl.ANY),
        scratch_shapes=[pltpu.SemaphoreType.DMA],
    )(goal_logits)


# speedup vs baseline: 27.8154x; 27.8154x over previous
import jax
import jax.numpy as jnp
from jax.experimental import pallas as pl

_N = 1_000_000


def _copy_body(in_ref, out_ref):
    out_ref[...] = in_ref[...]


def kernel(goal_logits):
    return pl.pallas_call(
        _copy_body,
        out_shape=jax.ShapeDtypeStruct((_N,), jnp.float32),
    )(goal_logits)
